# baseline (speedup 1.0000x reference)
import functools

import jax
import jax.numpy as jnp
from jax import lax
from jax.experimental import pallas as pl
from jax.experimental.pallas import tpu as pltpu
from jax.experimental.pallas import tpu_sc as plsc

N = 10000
E = 320000
D = 128
NPAD = 10240
NC = 2
NS = 16
NW = NC * NS
EPW = E // NW
BB = 128
NB = EPW // BB
TT = EPW - NB * BB
ZPT = NPAD // NS
RB = 1024
FB = 1000

_mesh = plsc.VectorSubcoreMesh(core_axis_name="c", subcore_axis_name="s")


@functools.partial(
    pl.kernel,
    out_type=jax.ShapeDtypeStruct((NC, NPAD), jnp.float32),
    mesh=_mesh,
    scratch_types=[
        pltpu.VMEM((BB,), jnp.int32),
        pltpu.VMEM((BB,), jnp.int32),
        pltpu.VMEM((TT,), jnp.int32),
        pltpu.VMEM((BB,), jnp.float32),
        pltpu.VMEM((ZPT,), jnp.float32),
        pltpu.VMEM_SHARED((NPAD,), jnp.float32),
        pltpu.SemaphoreType.DMA,
        pltpu.SemaphoreType.DMA,
        pltpu.SemaphoreType.DMA,
        pltpu.SemaphoreType.DMA,
    ],
)
def _deg_kernel(dst_hbm, deg_out, id0, id1, idt, ones_v, zb_v, sdeg,
                sem_i0, sem_i1, sem_s0, sem_s1):
    cid = lax.axis_index("c")
    sid = lax.axis_index("s")
    wid = sid * NC + cid
    base = wid * EPW

    def id_start(i, buf, sem):
        pltpu.async_copy(dst_hbm.at[pl.ds(base + i * BB, BB)], buf, sem)

    def id_wait(sem):
        pltpu.make_async_copy(dst_hbm.at[pl.ds(base, BB)], id0, sem).wait()

    def scatter_start(buf, sem):
        pltpu.async_copy(ones_v, sdeg.at[buf], sem, add=True)

    def scatter_wait(sem):
        pltpu.make_async_copy(ones_v, sdeg.at[id0], sem).wait()

    id_start(0, id0, sem_i0)

    def fill(i, _):
        ones_v[pl.ds(i * 16, 16)] = jnp.full((16,), 1.0, jnp.float32)
        return 0
    lax.fori_loop(0, BB // 16, fill, 0)

    def fill0(i, _):
        zb_v[pl.ds(i * 16, 16)] = jnp.zeros((16,), jnp.float32)
        return 0
    lax.fori_loop(0, ZPT // 16, fill0, 0)

    pltpu.sync_copy(zb_v, sdeg.at[pl.ds(sid * ZPT, ZPT)])
    plsc.subcore_barrier()

    def pair(k, _):
        i0 = 2 * k
        id_wait(sem_i0)
        scatter_start(id0, sem_s0)

        @pl.when(k > 0)
        def _():
            scatter_wait(sem_s1)
        id_start(i0 + 1, id1, sem_i1)
        id_wait(sem_i1)
        scatter_start(id1, sem_s1)
        scatter_wait(sem_s0)

        @pl.when(i0 + 2 < NB)
        def _():
            id_start(i0 + 2, id0, sem_i0)
        return 0
    lax.fori_loop(0, NB // 2, pair, 0)

    pltpu.sync_copy(dst_hbm.at[pl.ds(base + NB * BB, TT)], idt)
    pltpu.async_copy(ones_v.at[pl.ds(0, TT)], sdeg.at[idt], sem_s0, add=True)
    pltpu.make_async_copy(ones_v.at[pl.ds(0, TT)], sdeg.at[idt], sem_s0).wait()
    scatter_wait(sem_s1)

    plsc.subcore_barrier()
    pltpu.sync_copy(sdeg.at[pl.ds(sid * ZPT, ZPT)],
                    deg_out.at[cid, pl.ds(sid * ZPT, ZPT)])


@functools.partial(
    pl.kernel,
    out_type=jax.ShapeDtypeStruct((NC, NPAD, D), jnp.float32),
    mesh=_mesh,
    scratch_types=[
        pltpu.VMEM((BB,), jnp.int32),
        pltpu.VMEM((BB,), jnp.int32),
        pltpu.VMEM((BB,), jnp.int32),
        pltpu.VMEM((BB,), jnp.int32),
        pltpu.VMEM((TT,), jnp.int32),
        pltpu.VMEM((TT,), jnp.int32),
        pltpu.VMEM((BB, D), jnp.float32),
        pltpu.VMEM((BB, D), jnp.float32),
        pltpu.VMEM((TT, D), jnp.float32),
        pltpu.VMEM_SHARED((NPAD, D), jnp.float32),
        pltpu.SemaphoreType.DMA,
        pltpu.SemaphoreType.DMA,
        pltpu.SemaphoreType.DMA,
        pltpu.SemaphoreType.DMA,
        pltpu.SemaphoreType.DMA,
        pltpu.SemaphoreType.DMA,
        pltpu.SemaphoreType.DMA,
    ],
)
def _prop_kernel(g_hbm, src_hbm, dst_hbm, parts_out,
                 is0, is1, id0, id1, ist, idt, r0, r1, rt, sacc,
                 sem_g, sem_s0, sem_s1, sem_is0, sem_is1, sem_id0, sem_id1):
    cid = lax.axis_index("c")
    sid = lax.axis_index("s")
    wid = sid * NC + cid
    base = wid * EPW

    def is_start(i, buf, sem):
        pltpu.async_copy(src_hbm.at[pl.ds(base + i * BB, BB)], buf, sem)

    def id_start(i, buf, sem):
        pltpu.async_copy(dst_hbm.at[pl.ds(base + i * BB, BB)], buf, sem)

    def idx_wait(sem):
        pltpu.make_async_copy(src_hbm.at[pl.ds(base, BB)], is0, sem).wait()

    def gather_start(buf_i, buf_r):
        pltpu.async_copy(g_hbm.at[buf_i], buf_r, sem_g)

    def gather_wait():
        pltpu.make_async_copy(g_hbm.at[is0], r0, sem_g).wait()

    def scatter_start(buf_i, buf_r, sem):
        pltpu.async_copy(buf_r, sacc.at[buf_i], sem, add=True)

    def scatter_wait(sem):
        pltpu.make_async_copy(r0, sacc.at[id0], sem).wait()

    is_start(0, is0, sem_is0)
    id_start(0, id0, sem_id0)

    zbase = sid * ZPT

    @pl.when(cid == 0)
    def _():
        for off in (0, 128, 256, 384, 512):
            pltpu.sync_copy(g_hbm.at[pl.ds(zbase + off, BB)],
                            sacc.at[pl.ds(zbase + off, BB)])

    @pl.when(cid != 0)
    def _():
        def fill0(i, _):
            r1[i // 8, pl.ds((i % 8) * 16, 16)] = jnp.zeros((16,),
                                                            jnp.float32)
            return 0
        lax.fori_loop(0, BB * (D // 16), fill0, 0)
        for off in (0, 128, 256, 384, 512):
            pltpu.sync_copy(r1, sacc.at[pl.ds(zbase + off, BB)])
    plsc.subcore_barrier()

    idx_wait(sem_is0)
    gather_start(is0, r0)
    is_start(1, is1, sem_is1)

    def pair(k, _):
        i0 = 2 * k
        gather_wait()

        @pl.when(i0 + 2 < NB)
        def _():
            is_start(i0 + 2, is0, sem_is0)
        idx_wait(sem_id0)
        scatter_start(id0, r0, sem_s0)
        idx_wait(sem_is1)

        @pl.when(k > 0)
        def _():
            scatter_wait(sem_s1)
        id_start(i0 + 1, id1, sem_id1)
        gather_start(is1, r1)
        gather_wait()

        @pl.when(i0 + 3 < NB)
        def _():
            is_start(i0 + 3, is1, sem_is1)
        idx_wait(sem_id1)
        scatter_start(id1, r1, sem_s1)

        @pl.when(i0 + 2 < NB)
        def _():
            idx_wait(sem_is0)
        scatter_wait(sem_s0)

        @pl.when(i0 + 2 < NB)
        def _():
            id_start(i0 + 2, id0, sem_id0)
            gather_start(is0, r0)
        return 0
    lax.fori_loop(0, NB // 2, pair, 0)

    pltpu.sync_copy(src_hbm.at[pl.ds(base + NB * BB, TT)], ist)
    pltpu.sync_copy(dst_hbm.at[pl.ds(base + NB * BB, TT)], idt)
    pltpu.async_copy(g_hbm.at[ist], rt, sem_g)
    pltpu.make_async_copy(g_hbm.at[ist], rt, sem_g).wait()
    scatter_wait(sem_s1)
    pltpu.async_copy(rt, sacc.at[idt], sem_s0, add=True)
    pltpu.make_async_copy(rt, sacc.at[idt], sem_s0).wait()

    plsc.subcore_barrier()
    pltpu.sync_copy(sacc.at[pl.ds(sid * ZPT, ZPT)],
                    parts_out.at[cid, pl.ds(sid * ZPT, ZPT)])


REPACK_C = 32768


def _repack(ei):
    def body(ei_ref, s_ref, d_ref):
        s_ref[...] = ei_ref[0, :]
        d_ref[...] = ei_ref[1, :]

    return pl.pallas_call(
        body,
        grid=((E + REPACK_C - 1) // REPACK_C,),
        in_specs=[pl.BlockSpec((2, REPACK_C), lambda i: (0, i))],
        out_specs=[pl.BlockSpec((REPACK_C,), lambda i: (i,)),
                   pl.BlockSpec((REPACK_C,), lambda i: (i,))],
        out_shape=[jax.ShapeDtypeStruct((E,), jnp.int32),
                   jax.ShapeDtypeStruct((E,), jnp.int32)],
    )(ei)


def _transform(x_pad, W, degs):
    def body(x_ref, w_ref, deg_ref, g_ref):
        deg = deg_ref[0, :] + deg_ref[1, :] + 1.0
        dis = lax.rsqrt(deg)
        h = jnp.dot(x_ref[...], w_ref[...], preferred_element_type=jnp.float32)
        g_ref[...] = h * dis[:, None]

    return pl.pallas_call(
        body,
        grid=(NPAD // RB,),
        in_specs=[
            pl.BlockSpec((RB, D), lambda i: (i, 0)),
            pl.BlockSpec((D, D), lambda i: (0, 0)),
            pl.BlockSpec((NC, RB), lambda i: (0, i)),
        ],
        out_specs=pl.BlockSpec((RB, D), lambda i: (i, 0)),
        out_shape=jax.ShapeDtypeStruct((NPAD, D), jnp.float32),
    )(x_pad, W, degs)


def _finalize(parts, degs3, b):
    def body(p_ref, deg_ref, b_ref, o_ref):
        deg = deg_ref[0] + deg_ref[1] + 1.0
        dis = lax.rsqrt(deg)
        s = p_ref[0] + p_ref[1]
        o_ref[...] = jnp.tanh(s * dis + b_ref[...][None, :])

    return pl.pallas_call(
        body,
        grid=(N // FB,),
        in_specs=[
            pl.BlockSpec((NC, FB, D), lambda i: (0, i, 0)),
            pl.BlockSpec((NC, FB, 1), lambda i: (0, i, 0)),
            pl.BlockSpec((D,), lambda i: (0,)),
        ],
        out_specs=pl.BlockSpec((FB, D), lambda i: (i, 0)),
        out_shape=jax.ShapeDtypeStruct((N, D), jnp.float32),
    )(parts, degs3, b)


def kernel(x, edge_index, W, b):
    ei = edge_index.astype(jnp.int32)
    x_pad = jnp.pad(x, ((0, NPAD - N), (0, 0)))

    src, dst = _repack(ei)
    degs = _deg_kernel(dst)
    g = _transform(x_pad, W, degs)
    parts = _prop_kernel(g, src, dst)
    return _finalize(parts, degs[:, :N, None], b)


# speedup vs baseline: 1.1675x; 1.1675x over previous
import functools

import jax
import jax.numpy as jnp
from jax import lax
from jax.experimental import pallas as pl
from jax.experimental.pallas import tpu as pltpu
from jax.experimental.pallas import tpu_sc as plsc

N = 10000
E = 320000
D = 128
NPAD = 10240
NC = 2
NS = 16
NW = NC * NS
EPW = E // NW
BB = 128
NB = EPW // BB
TT = EPW - NB * BB
ZPT = NPAD // NS
RB = 1024
FB = 1000

_mesh = plsc.VectorSubcoreMesh(core_axis_name="c", subcore_axis_name="s")


DEG_SLOTS = 8
DEG_WIN = 4


@functools.partial(
    pl.kernel,
    out_type=jax.ShapeDtypeStruct((NC, NPAD), jnp.float32),
    mesh=_mesh,
    scratch_types=[
        pltpu.VMEM((NB * BB,), jnp.int32),
        pltpu.VMEM((DEG_SLOTS, BB), jnp.int32),
        pltpu.VMEM((TT,), jnp.int32),
        pltpu.VMEM((BB,), jnp.float32),
        pltpu.VMEM((ZPT,), jnp.float32),
        pltpu.VMEM_SHARED((NPAD,), jnp.float32),
        pltpu.SemaphoreType.DMA,
        pltpu.SemaphoreType.DMA,
    ],
)
def _deg_kernel(dst_hbm, deg_out, idv, slots, idt, ones_v, zb_v, sdeg,
                sem_p, sem_s):
    cid = lax.axis_index("c")
    sid = lax.axis_index("s")
    wid = sid * NC + cid
    base = wid * EPW

    pltpu.async_copy(dst_hbm.at[pl.ds(base, NB * BB)], idv, sem_p)

    def fill(i, _):
        ones_v[pl.ds(i * 16, 16)] = jnp.full((16,), 1.0, jnp.float32)
        return 0
    lax.fori_loop(0, BB // 16, fill, 0)

    def fill0(i, _):
        zb_v[pl.ds(i * 16, 16)] = jnp.zeros((16,), jnp.float32)
        return 0
    lax.fori_loop(0, ZPT // 16, fill0, 0)

    pltpu.sync_copy(zb_v, sdeg.at[pl.ds(sid * ZPT, ZPT)])
    pltpu.make_async_copy(dst_hbm.at[pl.ds(base, NB * BB)], idv, sem_p).wait()
    plsc.subcore_barrier()

    def scatter_wait():
        pltpu.make_async_copy(ones_v, sdeg.at[slots.at[0]], sem_s).wait()

    def step(i, _):
        b = lax.rem(i, DEG_SLOTS)
        for j in range(BB // 16):
            slots[b, pl.ds(j * 16, 16)] = idv[pl.ds(i * BB + j * 16, 16)]
        pltpu.async_copy(ones_v, sdeg.at[slots.at[b]], sem_s, add=True)

        @pl.when(i >= DEG_WIN)
        def _():
            scatter_wait()
        return 0
    lax.fori_loop(0, NB, step, 0)
    lax.fori_loop(0, DEG_WIN, lambda i, _: (scatter_wait(), 0)[1], 0)

    pltpu.sync_copy(dst_hbm.at[pl.ds(base + NB * BB, TT)], idt)
    pltpu.async_copy(ones_v.at[pl.ds(0, TT)], sdeg.at[idt], sem_s, add=True)
    pltpu.make_async_copy(ones_v.at[pl.ds(0, TT)], sdeg.at[idt], sem_s).wait()

    plsc.subcore_barrier()
    pltpu.sync_copy(sdeg.at[pl.ds(sid * ZPT, ZPT)],
                    deg_out.at[cid, pl.ds(sid * ZPT, ZPT)])


@functools.partial(
    pl.kernel,
    out_type=jax.ShapeDtypeStruct((NC, NPAD, D), jnp.float32),
    mesh=_mesh,
    scratch_types=[
        pltpu.VMEM((BB,), jnp.int32),
        pltpu.VMEM((BB,), jnp.int32),
        pltpu.VMEM((BB,), jnp.int32),
        pltpu.VMEM((BB,), jnp.int32),
        pltpu.VMEM((TT,), jnp.int32),
        pltpu.VMEM((TT,), jnp.int32),
        pltpu.VMEM((BB, D), jnp.float32),
        pltpu.VMEM((BB, D), jnp.float32),
        pltpu.VMEM((TT, D), jnp.float32),
        pltpu.VMEM_SHARED((NPAD, D), jnp.float32),
        pltpu.SemaphoreType.DMA,
        pltpu.SemaphoreType.DMA,
        pltpu.SemaphoreType.DMA,
        pltpu.SemaphoreType.DMA,
        pltpu.SemaphoreType.DMA,
        pltpu.SemaphoreType.DMA,
        pltpu.SemaphoreType.DMA,
    ],
)
def _prop_kernel(g_hbm, src_hbm, dst_hbm, parts_out,
                 is0, is1, id0, id1, ist, idt, r0, r1, rt, sacc,
                 sem_g, sem_s0, sem_s1, sem_is0, sem_is1, sem_id0, sem_id1):
    cid = lax.axis_index("c")
    sid = lax.axis_index("s")
    wid = sid * NC + cid
    base = wid * EPW

    def is_start(i, buf, sem):
        pltpu.async_copy(src_hbm.at[pl.ds(base + i * BB, BB)], buf, sem)

    def id_start(i, buf, sem):
        pltpu.async_copy(dst_hbm.at[pl.ds(base + i * BB, BB)], buf, sem)

    def idx_wait(sem):
        pltpu.make_async_copy(src_hbm.at[pl.ds(base, BB)], is0, sem).wait()

    def gather_start(buf_i, buf_r):
        pltpu.async_copy(g_hbm.at[buf_i], buf_r, sem_g)

    def gather_wait():
        pltpu.make_async_copy(g_hbm.at[is0], r0, sem_g).wait()

    def scatter_start(buf_i, buf_r, sem):
        pltpu.async_copy(buf_r, sacc.at[buf_i], sem, add=True)

    def scatter_wait(sem):
        pltpu.make_async_copy(r0, sacc.at[id0], sem).wait()

    is_start(0, is0, sem_is0)
    id_start(0, id0, sem_id0)

    zbase = sid * ZPT

    @pl.when(cid == 0)
    def _():
        for off in (0, 128, 256, 384, 512):
            pltpu.sync_copy(g_hbm.at[pl.ds(zbase + off, BB)],
                            sacc.at[pl.ds(zbase + off, BB)])

    @pl.when(cid != 0)
    def _():
        def fill0(i, _):
            r1[i // 8, pl.ds((i % 8) * 16, 16)] = jnp.zeros((16,),
                                                            jnp.float32)
            return 0
        lax.fori_loop(0, BB * (D // 16), fill0, 0)
        for off in (0, 128, 256, 384, 512):
            pltpu.sync_copy(r1, sacc.at[pl.ds(zbase + off, BB)])
    plsc.subcore_barrier()

    idx_wait(sem_is0)
    gather_start(is0, r0)
    is_start(1, is1, sem_is1)

    def pair(k, _):
        i0 = 2 * k
        gather_wait()

        @pl.when(i0 + 2 < NB)
        def _():
            is_start(i0 + 2, is0, sem_is0)
        idx_wait(sem_id0)
        scatter_start(id0, r0, sem_s0)
        idx_wait(sem_is1)

        @pl.when(k > 0)
        def _():
            scatter_wait(sem_s1)
        id_start(i0 + 1, id1, sem_id1)
        gather_start(is1, r1)
        gather_wait()

        @pl.when(i0 + 3 < NB)
        def _():
            is_start(i0 + 3, is1, sem_is1)
        idx_wait(sem_id1)
        scatter_start(id1, r1, sem_s1)

        @pl.when(i0 + 2 < NB)
        def _():
            idx_wait(sem_is0)
        scatter_wait(sem_s0)

        @pl.when(i0 + 2 < NB)
        def _():
            id_start(i0 + 2, id0, sem_id0)
            gather_start(is0, r0)
        return 0
    lax.fori_loop(0, NB // 2, pair, 0)

    pltpu.sync_copy(src_hbm.at[pl.ds(base + NB * BB, TT)], ist)
    pltpu.sync_copy(dst_hbm.at[pl.ds(base + NB * BB, TT)], idt)
    pltpu.async_copy(g_hbm.at[ist], rt, sem_g)
    pltpu.make_async_copy(g_hbm.at[ist], rt, sem_g).wait()
    scatter_wait(sem_s1)
    pltpu.async_copy(rt, sacc.at[idt], sem_s0, add=True)
    pltpu.make_async_copy(rt, sacc.at[idt], sem_s0).wait()

    plsc.subcore_barrier()
    pltpu.sync_copy(sacc.at[pl.ds(sid * ZPT, ZPT)],
                    parts_out.at[cid, pl.ds(sid * ZPT, ZPT)])


REPACK_C = 32768


def _repack(ei):
    def body(ei_ref, s_ref, d_ref):
        s_ref[...] = ei_ref[0, :]
        d_ref[...] = ei_ref[1, :]

    return pl.pallas_call(
        body,
        grid=((E + REPACK_C - 1) // REPACK_C,),
        in_specs=[pl.BlockSpec((2, REPACK_C), lambda i: (0, i))],
        out_specs=[pl.BlockSpec((REPACK_C,), lambda i: (i,)),
                   pl.BlockSpec((REPACK_C,), lambda i: (i,))],
        out_shape=[jax.ShapeDtypeStruct((E,), jnp.int32),
                   jax.ShapeDtypeStruct((E,), jnp.int32)],
    )(ei)


def _transform(x_pad, W, degs):
    def body(x_ref, w_ref, deg_ref, g_ref):
        deg = deg_ref[0, :] + deg_ref[1, :] + 1.0
        dis = lax.rsqrt(deg)
        h = jnp.dot(x_ref[...], w_ref[...], preferred_element_type=jnp.float32)
        g_ref[...] = h * dis[:, None]

    return pl.pallas_call(
        body,
        grid=(NPAD // RB,),
        in_specs=[
            pl.BlockSpec((RB, D), lambda i: (i, 0)),
            pl.BlockSpec((D, D), lambda i: (0, 0)),
            pl.BlockSpec((NC, RB), lambda i: (0, i)),
        ],
        out_specs=pl.BlockSpec((RB, D), lambda i: (i, 0)),
        out_shape=jax.ShapeDtypeStruct((NPAD, D), jnp.float32),
    )(x_pad, W, degs)


def _finalize(parts, degs3, b):
    def body(p_ref, deg_ref, b_ref, o_ref):
        deg = deg_ref[0] + deg_ref[1] + 1.0
        dis = lax.rsqrt(deg)
        s = p_ref[0] + p_ref[1]
        o_ref[...] = jnp.tanh(s * dis + b_ref[...][None, :])

    return pl.pallas_call(
        body,
        grid=(N // FB,),
        in_specs=[
            pl.BlockSpec((NC, FB, D), lambda i: (0, i, 0)),
            pl.BlockSpec((NC, FB, 1), lambda i: (0, i, 0)),
            pl.BlockSpec((D,), lambda i: (0,)),
        ],
        out_specs=pl.BlockSpec((FB, D), lambda i: (i, 0)),
        out_shape=jax.ShapeDtypeStruct((N, D), jnp.float32),
    )(parts, degs3, b)


def kernel(x, edge_index, W, b):
    ei = edge_index.astype(jnp.int32)
    x_pad = jnp.pad(x, ((0, NPAD - N), (0, 0)))

    src, dst = _repack(ei)
    degs = _deg_kernel(dst)
    g = _transform(x_pad, W, degs)
    parts = _prop_kernel(g, src, dst)
    return _finalize(parts, degs[:, :N, None], b)


# speedup vs baseline: 1.1851x; 1.0151x over previous
import functools

import jax
import jax.numpy as jnp
from jax import lax
from jax.experimental import pallas as pl
from jax.experimental.pallas import tpu as pltpu
from jax.experimental.pallas import tpu_sc as plsc

N = 10000
E = 320000
D = 128
NPAD = 10240
NC = 2
NS = 16
NW = NC * NS
BB = 128
NB = 78
EPW = NB * BB
XB = E // BB - NW * NB
XBASE = NW * EPW
ZPT = NPAD // NS
RB = 1024
FB = 1000
DEG_SLOTS = 8
DEG_WIN = 4

_mesh = plsc.VectorSubcoreMesh(core_axis_name="c", subcore_axis_name="s")


@functools.partial(
    pl.kernel,
    out_type=jax.ShapeDtypeStruct((NC, NPAD), jnp.float32),
    mesh=_mesh,
    scratch_types=[
        pltpu.VMEM((2, EPW), jnp.int32),
        pltpu.VMEM((DEG_SLOTS, BB), jnp.int32),
        pltpu.VMEM((2, BB), jnp.int32),
        pltpu.VMEM((BB,), jnp.float32),
        pltpu.VMEM((ZPT,), jnp.float32),
        pltpu.VMEM_SHARED((NPAD,), jnp.float32),
        pltpu.SemaphoreType.DMA,
        pltpu.SemaphoreType.DMA,
    ],
)
def _deg_kernel(ei_hbm, deg_out, idv, slots, ixt, ones_v, zb_v, sdeg,
                sem_p, sem_s):
    cid = lax.axis_index("c")
    sid = lax.axis_index("s")
    wid = sid * NC + cid
    base = wid * EPW

    pltpu.async_copy(ei_hbm.at[:, pl.ds(base, EPW)], idv, sem_p)

    def fill(i, _):
        ones_v[pl.ds(i * 16, 16)] = jnp.full((16,), 1.0, jnp.float32)
        return 0
    lax.fori_loop(0, BB // 16, fill, 0)

    def fill0(i, _):
        zb_v[pl.ds(i * 16, 16)] = jnp.zeros((16,), jnp.float32)
        return 0
    lax.fori_loop(0, ZPT // 16, fill0, 0)

    pltpu.sync_copy(zb_v, sdeg.at[pl.ds(sid * ZPT, ZPT)])
    pltpu.make_async_copy(ei_hbm.at[:, pl.ds(base, EPW)], idv, sem_p).wait()
    plsc.subcore_barrier()

    def scatter_wait():
        pltpu.make_async_copy(ones_v, sdeg.at[slots.at[0]], sem_s).wait()

    def step(i, _):
        b = lax.rem(i, DEG_SLOTS)
        for j in range(BB // 16):
            slots[b, pl.ds(j * 16, 16)] = idv[1, pl.ds(i * BB + j * 16, 16)]
        pltpu.async_copy(ones_v, sdeg.at[slots.at[b]], sem_s, add=True)

        @pl.when(i >= DEG_WIN)
        def _():
            scatter_wait()
        return 0
    lax.fori_loop(0, NB, step, 0)
    lax.fori_loop(0, DEG_WIN, lambda i, _: (scatter_wait(), 0)[1], 0)

    @pl.when(wid < XB)
    def _():
        pltpu.sync_copy(ei_hbm.at[:, pl.ds(XBASE + wid * BB, BB)], ixt)
        pltpu.async_copy(ones_v, sdeg.at[ixt.at[1]], sem_s, add=True)
        pltpu.make_async_copy(ones_v, sdeg.at[ixt.at[1]], sem_s).wait()

    plsc.subcore_barrier()
    pltpu.sync_copy(sdeg.at[pl.ds(sid * ZPT, ZPT)],
                    deg_out.at[cid, pl.ds(sid * ZPT, ZPT)])


@functools.partial(
    pl.kernel,
    out_type=jax.ShapeDtypeStruct((NC, NPAD, D), jnp.float32),
    mesh=_mesh,
    scratch_types=[
        pltpu.VMEM((8, BB), jnp.int32),
        pltpu.VMEM((2, BB, D), jnp.float32),
        pltpu.VMEM((2, BB), jnp.int32),
        pltpu.VMEM_SHARED((NPAD, D), jnp.float32),
        pltpu.SemaphoreType.DMA,
        pltpu.SemaphoreType.DMA,
        pltpu.SemaphoreType.DMA,
    ],
)
def _prop_kernel(g_hbm, ei_hbm, parts_out, i2, rows, ixt, sacc,
                 sem_i, sem_g, sem_s):
    cid = lax.axis_index("c")
    sid = lax.axis_index("s")
    wid = sid * NC + cid
    base = wid * EPW

    def i2_start(i):
        s = lax.rem(i, 4)
        pltpu.async_copy(ei_hbm.at[:, pl.ds(base + i * BB, BB)],
                         i2.at[pl.ds(2 * s, 2)], sem_i)

    def i2_wait():
        pltpu.make_async_copy(ei_hbm.at[:, pl.ds(base, BB)],
                              i2.at[pl.ds(0, 2)], sem_i).wait()

    def gather_start(i, b):
        pltpu.async_copy(g_hbm.at[i2.at[2 * lax.rem(i, 4)]], rows.at[b],
                         sem_g)

    def gather_wait():
        pltpu.make_async_copy(g_hbm.at[i2.at[0]], rows.at[0], sem_g).wait()

    def scatter_start(i, b):
        pltpu.async_copy(rows.at[b], sacc.at[i2.at[2 * lax.rem(i, 4) + 1]],
                         sem_s, add=True)

    def scatter_wait():
        pltpu.make_async_copy(rows.at[0], sacc.at[i2.at[1]], sem_s).wait()

    i2_start(0)
    i2_start(1)

    zbase = sid * ZPT

    @pl.when(cid == 0)
    def _():
        for off in (0, 128, 256, 384, 512):
            pltpu.sync_copy(g_hbm.at[pl.ds(zbase + off, BB)],
                            sacc.at[pl.ds(zbase + off, BB)])

    @pl.when(cid != 0)
    def _():
        def fill0(i, _):
            rows[1, i // 8, pl.ds((i % 8) * 16, 16)] = jnp.zeros((16,),
                                                                 jnp.float32)
            return 0
        lax.fori_loop(0, BB * (D // 16), fill0, 0)
        for off in (0, 128, 256, 384, 512):
            pltpu.sync_copy(rows.at[1], sacc.at[pl.ds(zbase + off, BB)])
    plsc.subcore_barrier()

    i2_wait()
    gather_start(0, 0)

    def step(i, _):
        b = lax.rem(i, 2)
        gather_wait()
        scatter_start(i, b)

        @pl.when(i >= 1)
        def _():
            scatter_wait()

        @pl.when(i + 2 < NB)
        def _():
            i2_start(i + 2)

        @pl.when(i + 1 < NB)
        def _():
            i2_wait()
            gather_start(i + 1, 1 - b)
        return 0
    lax.fori_loop(0, NB, step, 0)
    scatter_wait()

    @pl.when(wid < XB)
    def _():
        pltpu.sync_copy(ei_hbm.at[:, pl.ds(XBASE + wid * BB, BB)], ixt)
        pltpu.async_copy(g_hbm.at[ixt.at[0]], rows.at[0], sem_g)
        pltpu.make_async_copy(g_hbm.at[ixt.at[0]], rows.at[0], sem_g).wait()
        pltpu.async_copy(rows.at[0], sacc.at[ixt.at[1]], sem_s, add=True)
        pltpu.make_async_copy(rows.at[0], sacc.at[ixt.at[1]], sem_s).wait()

    plsc.subcore_barrier()
    pltpu.sync_copy(sacc.at[pl.ds(sid * ZPT, ZPT)],
                    parts_out.at[cid, pl.ds(sid * ZPT, ZPT)])


def _transform(x_pad, W, degs):
    def body(x_ref, w_ref, deg_ref, g_ref):
        deg = deg_ref[0, :] + deg_ref[1, :] + 1.0
        dis = lax.rsqrt(deg)
        h = jnp.dot(x_ref[...], w_ref[...], preferred_element_type=jnp.float32)
        g_ref[...] = h * dis[:, None]

    return pl.pallas_call(
        body,
        grid=(NPAD // RB,),
        in_specs=[
            pl.BlockSpec((RB, D), lambda i: (i, 0)),
            pl.BlockSpec((D, D), lambda i: (0, 0)),
            pl.BlockSpec((NC, RB), lambda i: (0, i)),
        ],
        out_specs=pl.BlockSpec((RB, D), lambda i: (i, 0)),
        out_shape=jax.ShapeDtypeStruct((NPAD, D), jnp.float32),
    )(x_pad, W, degs)


def _finalize(parts, degs3, b):
    def body(p_ref, deg_ref, b_ref, o_ref):
        deg = deg_ref[0] + deg_ref[1] + 1.0
        dis = lax.rsqrt(deg)
        s = p_ref[0] + p_ref[1]
        o_ref[...] = jnp.tanh(s * dis + b_ref[...][None, :])

    return pl.pallas_call(
        body,
        grid=(N // FB,),
        in_specs=[
            pl.BlockSpec((NC, FB, D), lambda i: (0, i, 0)),
            pl.BlockSpec((NC, FB, 1), lambda i: (0, i, 0)),
            pl.BlockSpec((D,), lambda i: (0,)),
        ],
        out_specs=pl.BlockSpec((FB, D), lambda i: (i, 0)),
        out_shape=jax.ShapeDtypeStruct((N, D), jnp.float32),
    )(parts, degs3, b)


def kernel(x, edge_index, W, b):
    ei = edge_index.astype(jnp.int32)
    x_pad = jnp.pad(x, ((0, NPAD - N), (0, 0)))

    degs = _deg_kernel(ei)
    g = _transform(x_pad, W, degs)
    parts = _prop_kernel(g, ei)
    return _finalize(parts, degs[:, :N, None], b)


# speedup vs baseline: 1.2056x; 1.0173x over previous
import functools

import jax
import jax.numpy as jnp
from jax import lax
from jax.experimental import pallas as pl
from jax.experimental.pallas import tpu as pltpu
from jax.experimental.pallas import tpu_sc as plsc

N = 10000
E = 320000
D = 128
NPAD = 10240
NC = 2
NS = 16
NW = NC * NS
BB = 128
NB = 78
EPW = NB * BB
XB = E // BB - NW * NB
XBASE = NW * EPW
ZPT = NPAD // NS
RB = 2048
FB = 2000
DEG_SLOTS = 8
DEG_WIN = 4

_mesh = plsc.VectorSubcoreMesh(core_axis_name="c", subcore_axis_name="s")


@functools.partial(
    pl.kernel,
    out_type=jax.ShapeDtypeStruct((NC, NPAD), jnp.float32),
    mesh=_mesh,
    scratch_types=[
        pltpu.VMEM((2, EPW), jnp.int32),
        pltpu.VMEM((DEG_SLOTS, BB), jnp.int32),
        pltpu.VMEM((2, BB), jnp.int32),
        pltpu.VMEM((BB,), jnp.float32),
        pltpu.VMEM((ZPT,), jnp.float32),
        pltpu.VMEM_SHARED((NPAD,), jnp.float32),
        pltpu.SemaphoreType.DMA,
        pltpu.SemaphoreType.DMA,
    ],
)
def _deg_kernel(ei_hbm, deg_out, idv, slots, ixt, ones_v, zb_v, sdeg,
                sem_p, sem_s):
    cid = lax.axis_index("c")
    sid = lax.axis_index("s")
    wid = sid * NC + cid
    base = wid * EPW

    pltpu.async_copy(ei_hbm.at[:, pl.ds(base, EPW)], idv, sem_p)

    def fill(i, _):
        ones_v[pl.ds(i * 16, 16)] = jnp.full((16,), 1.0, jnp.float32)
        return 0
    lax.fori_loop(0, BB // 16, fill, 0)

    def fill0(i, _):
        zb_v[pl.ds(i * 16, 16)] = jnp.zeros((16,), jnp.float32)
        return 0
    lax.fori_loop(0, ZPT // 16, fill0, 0)

    pltpu.sync_copy(zb_v, sdeg.at[pl.ds(sid * ZPT, ZPT)])
    pltpu.make_async_copy(ei_hbm.at[:, pl.ds(base, EPW)], idv, sem_p).wait()
    plsc.subcore_barrier()

    def scatter_wait():
        pltpu.make_async_copy(ones_v, sdeg.at[slots.at[0]], sem_s).wait()

    def step(i, _):
        b = lax.rem(i, DEG_SLOTS)
        for j in range(BB // 16):
            slots[b, pl.ds(j * 16, 16)] = idv[1, pl.ds(i * BB + j * 16, 16)]
        pltpu.async_copy(ones_v, sdeg.at[slots.at[b]], sem_s, add=True)

        @pl.when(i >= DEG_WIN)
        def _():
            scatter_wait()
        return 0
    lax.fori_loop(0, NB, step, 0)
    lax.fori_loop(0, DEG_WIN, lambda i, _: (scatter_wait(), 0)[1], 0)

    @pl.when(wid < XB)
    def _():
        pltpu.sync_copy(ei_hbm.at[:, pl.ds(XBASE + wid * BB, BB)], ixt)
        pltpu.async_copy(ones_v, sdeg.at[ixt.at[1]], sem_s, add=True)
        pltpu.make_async_copy(ones_v, sdeg.at[ixt.at[1]], sem_s).wait()

    plsc.subcore_barrier()
    pltpu.sync_copy(sdeg.at[pl.ds(sid * ZPT, ZPT)],
                    deg_out.at[cid, pl.ds(sid * ZPT, ZPT)])


@functools.partial(
    pl.kernel,
    out_type=jax.ShapeDtypeStruct((NC, NPAD, D), jnp.float32),
    mesh=_mesh,
    scratch_types=[
        pltpu.VMEM((8, BB), jnp.int32),
        pltpu.VMEM((2, BB, D), jnp.float32),
        pltpu.VMEM((2, BB), jnp.int32),
        pltpu.VMEM_SHARED((NPAD, D), jnp.float32),
        pltpu.SemaphoreType.DMA,
        pltpu.SemaphoreType.DMA,
        pltpu.SemaphoreType.DMA,
    ],
)
def _prop_kernel(g_hbm, ei_hbm, parts_out, i2, rows, ixt, sacc,
                 sem_i, sem_g, sem_s):
    cid = lax.axis_index("c")
    sid = lax.axis_index("s")
    wid = sid * NC + cid
    base = wid * EPW

    def i2_start(i):
        s = lax.rem(i, 4)
        pltpu.async_copy(ei_hbm.at[:, pl.ds(base + i * BB, BB)],
                         i2.at[pl.ds(2 * s, 2)], sem_i)

    def i2_wait():
        pltpu.make_async_copy(ei_hbm.at[:, pl.ds(base, BB)],
                              i2.at[pl.ds(0, 2)], sem_i).wait()

    def gather_start(i, b):
        pltpu.async_copy(g_hbm.at[i2.at[2 * lax.rem(i, 4)]], rows.at[b],
                         sem_g)

    def gather_wait():
        pltpu.make_async_copy(g_hbm.at[i2.at[0]], rows.at[0], sem_g).wait()

    def scatter_start(i, b):
        pltpu.async_copy(rows.at[b], sacc.at[i2.at[2 * lax.rem(i, 4) + 1]],
                         sem_s, add=True)

    def scatter_wait():
        pltpu.make_async_copy(rows.at[0], sacc.at[i2.at[1]], sem_s).wait()

    i2_start(0)
    i2_start(1)

    zbase = sid * ZPT

    @pl.when(cid == 0)
    def _():
        for off in (0, 128, 256, 384, 512):
            pltpu.async_copy(g_hbm.at[pl.ds(zbase + off, BB)],
                             sacc.at[pl.ds(zbase + off, BB)], sem_g)
        for off in (0, 128, 256, 384, 512):
            pltpu.make_async_copy(g_hbm.at[pl.ds(zbase + off, BB)],
                                  sacc.at[pl.ds(zbase + off, BB)],
                                  sem_g).wait()

    @pl.when(cid != 0)
    def _():
        def fill0(i, _):
            rows[1, i // 8, pl.ds((i % 8) * 16, 16)] = jnp.zeros((16,),
                                                                 jnp.float32)
            return 0
        lax.fori_loop(0, BB * (D // 16), fill0, 0)
        for off in (0, 128, 256, 384, 512):
            pltpu.async_copy(rows.at[1], sacc.at[pl.ds(zbase + off, BB)],
                             sem_g)
        for off in (0, 128, 256, 384, 512):
            pltpu.make_async_copy(rows.at[1],
                                  sacc.at[pl.ds(zbase + off, BB)],
                                  sem_g).wait()
    plsc.subcore_barrier()

    i2_wait()
    gather_start(0, 0)

    def step(i, _):
        b = lax.rem(i, 2)
        gather_wait()
        scatter_start(i, b)

        @pl.when(i >= 1)
        def _():
            scatter_wait()

        @pl.when(i + 2 < NB)
        def _():
            i2_start(i + 2)

        @pl.when(i + 1 < NB)
        def _():
            i2_wait()
            gather_start(i + 1, 1 - b)
        return 0
    lax.fori_loop(0, NB, step, 0)
    scatter_wait()

    @pl.when(wid < XB)
    def _():
        pltpu.sync_copy(ei_hbm.at[:, pl.ds(XBASE + wid * BB, BB)], ixt)
        pltpu.async_copy(g_hbm.at[ixt.at[0]], rows.at[0], sem_g)
        pltpu.make_async_copy(g_hbm.at[ixt.at[0]], rows.at[0], sem_g).wait()
        pltpu.async_copy(rows.at[0], sacc.at[ixt.at[1]], sem_s, add=True)
        pltpu.make_async_copy(rows.at[0], sacc.at[ixt.at[1]], sem_s).wait()

    plsc.subcore_barrier()
    pltpu.sync_copy(sacc.at[pl.ds(sid * ZPT, ZPT)],
                    parts_out.at[cid, pl.ds(sid * ZPT, ZPT)])


def _transform(x_pad, W, degs):
    def body(x_ref, w_ref, deg_ref, g_ref):
        deg = deg_ref[0, :] + deg_ref[1, :] + 1.0
        dis = lax.rsqrt(deg)
        h = jnp.dot(x_ref[...], w_ref[...], preferred_element_type=jnp.float32)
        g_ref[...] = h * dis[:, None]

    return pl.pallas_call(
        body,
        grid=(NPAD // RB,),
        in_specs=[
            pl.BlockSpec((RB, D), lambda i: (i, 0)),
            pl.BlockSpec((D, D), lambda i: (0, 0)),
            pl.BlockSpec((NC, RB), lambda i: (0, i)),
        ],
        out_specs=pl.BlockSpec((RB, D), lambda i: (i, 0)),
        out_shape=jax.ShapeDtypeStruct((NPAD, D), jnp.float32),
    )(x_pad, W, degs)


def _finalize(parts, degs3, b):
    def body(p_ref, deg_ref, b_ref, o_ref):
        deg = deg_ref[0] + deg_ref[1] + 1.0
        dis = lax.rsqrt(deg)
        s = p_ref[0] + p_ref[1]
        o_ref[...] = jnp.tanh(s * dis + b_ref[...][None, :])

    return pl.pallas_call(
        body,
        grid=(N // FB,),
        in_specs=[
            pl.BlockSpec((NC, FB, D), lambda i: (0, i, 0)),
            pl.BlockSpec((NC, FB, 1), lambda i: (0, i, 0)),
            pl.BlockSpec((D,), lambda i: (0,)),
        ],
        out_specs=pl.BlockSpec((FB, D), lambda i: (i, 0)),
        out_shape=jax.ShapeDtypeStruct((N, D), jnp.float32),
    )(parts, degs3, b)


def kernel(x, edge_index, W, b):
    ei = edge_index.astype(jnp.int32)
    x_pad = jnp.pad(x, ((0, NPAD - N), (0, 0)))

    degs = _deg_kernel(ei)
    g = _transform(x_pad, W, degs)
    parts = _prop_kernel(g, ei)
    return _finalize(parts, degs[:, :N, None], b)


# speedup vs baseline: 1.2116x; 1.0050x over previous
import functools

import jax
import jax.numpy as jnp
from jax import lax
from jax.experimental import pallas as pl
from jax.experimental.pallas import tpu as pltpu
from jax.experimental.pallas import tpu_sc as plsc

N = 10000
E = 320000
D = 128
NPAD = 10240
NC = 2
NS = 16
NW = NC * NS
BB = 128
NB = 78
EPW = NB * BB
XB = E // BB - NW * NB
XBASE = NW * EPW
ZPT = NPAD // NS
RB = 2048
FB = 2000
DEG_SLOTS = 8
DEG_WIN = 4

_mesh = plsc.VectorSubcoreMesh(core_axis_name="c", subcore_axis_name="s")


@functools.partial(
    pl.kernel,
    out_type=jax.ShapeDtypeStruct((NC, NPAD), jnp.float32),
    mesh=_mesh,
    scratch_types=[
        pltpu.VMEM((2, EPW), jnp.int32),
        pltpu.VMEM((DEG_SLOTS, BB), jnp.int32),
        pltpu.VMEM((2, BB), jnp.int32),
        pltpu.VMEM((BB,), jnp.float32),
        pltpu.VMEM((ZPT,), jnp.float32),
        pltpu.VMEM_SHARED((NPAD,), jnp.float32),
        pltpu.SemaphoreType.DMA,
        pltpu.SemaphoreType.DMA,
    ],
)
def _deg_kernel(ei_hbm, deg_out, idv, slots, ixt, ones_v, zb_v, sdeg,
                sem_p, sem_s):
    cid = lax.axis_index("c")
    sid = lax.axis_index("s")
    wid = sid * NC + cid
    base = wid * EPW

    pltpu.async_copy(ei_hbm.at[:, pl.ds(base, EPW)], idv, sem_p)

    def fill(i, _):
        ones_v[pl.ds(i * 16, 16)] = jnp.full((16,), 1.0, jnp.float32)
        return 0
    lax.fori_loop(0, BB // 16, fill, 0)

    def fill0(i, _):
        zb_v[pl.ds(i * 16, 16)] = jnp.zeros((16,), jnp.float32)
        return 0
    lax.fori_loop(0, ZPT // 16, fill0, 0)

    pltpu.sync_copy(zb_v, sdeg.at[pl.ds(sid * ZPT, ZPT)])
    pltpu.make_async_copy(ei_hbm.at[:, pl.ds(base, EPW)], idv, sem_p).wait()
    plsc.subcore_barrier()

    def scatter_wait():
        pltpu.make_async_copy(ones_v, sdeg.at[slots.at[0]], sem_s).wait()

    def step(i, _):
        b = lax.rem(i, DEG_SLOTS)
        for j in range(BB // 16):
            slots[b, pl.ds(j * 16, 16)] = idv[1, pl.ds(i * BB + j * 16, 16)]
        pltpu.async_copy(ones_v, sdeg.at[slots.at[b]], sem_s, add=True)

        @pl.when(i >= DEG_WIN)
        def _():
            scatter_wait()
        return 0
    lax.fori_loop(0, NB, step, 0)
    lax.fori_loop(0, DEG_WIN, lambda i, _: (scatter_wait(), 0)[1], 0)

    @pl.when(wid < XB)
    def _():
        pltpu.sync_copy(ei_hbm.at[:, pl.ds(XBASE + wid * BB, BB)], ixt)
        pltpu.async_copy(ones_v, sdeg.at[ixt.at[1]], sem_s, add=True)
        pltpu.make_async_copy(ones_v, sdeg.at[ixt.at[1]], sem_s).wait()

    plsc.subcore_barrier()
    pltpu.sync_copy(sdeg.at[pl.ds(sid * ZPT, ZPT)],
                    deg_out.at[cid, pl.ds(sid * ZPT, ZPT)])


@functools.partial(
    pl.kernel,
    out_type=jax.ShapeDtypeStruct((NC, NPAD, D), jnp.float32),
    mesh=_mesh,
    scratch_types=[
        pltpu.VMEM((8, BB), jnp.int32),
        pltpu.VMEM((2, BB, D), jnp.float32),
        pltpu.VMEM((2, BB), jnp.int32),
        pltpu.VMEM_SHARED((NPAD, D), jnp.float32),
        pltpu.SemaphoreType.DMA,
        pltpu.SemaphoreType.DMA,
        pltpu.SemaphoreType.DMA,
    ],
)
def _prop_kernel(g_hbm, ei_hbm, parts_out, i2, rows, ixt, sacc,
                 sem_i, sem_g, sem_s):
    cid = lax.axis_index("c")
    sid = lax.axis_index("s")
    wid = sid * NC + cid
    base = wid * EPW

    def i2_start(i):
        s = lax.rem(i, 4)
        pltpu.async_copy(ei_hbm.at[:, pl.ds(base + i * BB, BB)],
                         i2.at[pl.ds(2 * s, 2)], sem_i)

    def i2_wait():
        pltpu.make_async_copy(ei_hbm.at[:, pl.ds(base, BB)],
                              i2.at[pl.ds(0, 2)], sem_i).wait()

    def gather_start(i, b):
        pltpu.async_copy(g_hbm.at[i2.at[2 * lax.rem(i, 4)]], rows.at[b],
                         sem_g)

    def gather_wait():
        pltpu.make_async_copy(g_hbm.at[i2.at[0]], rows.at[0], sem_g).wait()

    def scatter_start(i, b):
        pltpu.async_copy(rows.at[b], sacc.at[i2.at[2 * lax.rem(i, 4) + 1]],
                         sem_s, add=True)

    def scatter_wait():
        pltpu.make_async_copy(rows.at[0], sacc.at[i2.at[1]], sem_s).wait()

    i2_start(0)
    i2_start(1)

    zbase = sid * ZPT

    @pl.when(cid == 0)
    def _():
        for off in (0, 128, 256, 384, 512):
            pltpu.async_copy(g_hbm.at[pl.ds(zbase + off, BB)],
                             sacc.at[pl.ds(zbase + off, BB)], sem_g)
        for off in (0, 128, 256, 384, 512):
            pltpu.make_async_copy(g_hbm.at[pl.ds(zbase + off, BB)],
                                  sacc.at[pl.ds(zbase + off, BB)],
                                  sem_g).wait()

    @pl.when(cid != 0)
    def _():
        def fill0(i, _):
            rows[1, i // 8, pl.ds((i % 8) * 16, 16)] = jnp.zeros((16,),
                                                                 jnp.float32)
            return 0
        lax.fori_loop(0, BB * (D // 16), fill0, 0)
        for off in (0, 128, 256, 384, 512):
            pltpu.async_copy(rows.at[1], sacc.at[pl.ds(zbase + off, BB)],
                             sem_g)
        for off in (0, 128, 256, 384, 512):
            pltpu.make_async_copy(rows.at[1],
                                  sacc.at[pl.ds(zbase + off, BB)],
                                  sem_g).wait()
    plsc.subcore_barrier()

    i2_wait()
    gather_start(0, 0)
    gather_wait()
    scatter_start(0, 0)
    i2_start(2)
    i2_wait()
    gather_start(1, 1)

    def step(i, _):
        b = lax.rem(i, 2)
        gather_wait()
        scatter_start(i, b)
        scatter_wait()
        i2_start(i + 2)
        i2_wait()
        gather_start(i + 1, 1 - b)
        return 0
    lax.fori_loop(1, NB - 2, step, 0)

    gather_wait()
    scatter_start(NB - 2, lax.rem(NB - 2, 2))
    scatter_wait()
    i2_wait()
    gather_start(NB - 1, lax.rem(NB - 1, 2))
    gather_wait()
    scatter_start(NB - 1, lax.rem(NB - 1, 2))
    scatter_wait()
    scatter_wait()

    @pl.when(wid < XB)
    def _():
        pltpu.sync_copy(ei_hbm.at[:, pl.ds(XBASE + wid * BB, BB)], ixt)
        pltpu.async_copy(g_hbm.at[ixt.at[0]], rows.at[0], sem_g)
        pltpu.make_async_copy(g_hbm.at[ixt.at[0]], rows.at[0], sem_g).wait()
        pltpu.async_copy(rows.at[0], sacc.at[ixt.at[1]], sem_s, add=True)
        pltpu.make_async_copy(rows.at[0], sacc.at[ixt.at[1]], sem_s).wait()

    plsc.subcore_barrier()
    pltpu.sync_copy(sacc.at[pl.ds(sid * ZPT, ZPT)],
                    parts_out.at[cid, pl.ds(sid * ZPT, ZPT)])


def _transform(x_pad, W, degs):
    def body(x_ref, w_ref, deg_ref, g_ref):
        deg = deg_ref[0, :] + deg_ref[1, :] + 1.0
        dis = lax.rsqrt(deg)
        h = jnp.dot(x_ref[...], w_ref[...], preferred_element_type=jnp.float32)
        g_ref[...] = h * dis[:, None]

    return pl.pallas_call(
        body,
        grid=(NPAD // RB,),
        in_specs=[
            pl.BlockSpec((RB, D), lambda i: (i, 0)),
            pl.BlockSpec((D, D), lambda i: (0, 0)),
            pl.BlockSpec((NC, RB), lambda i: (0, i)),
        ],
        out_specs=pl.BlockSpec((RB, D), lambda i: (i, 0)),
        out_shape=jax.ShapeDtypeStruct((NPAD, D), jnp.float32),
    )(x_pad, W, degs)


def _finalize(parts, degs3, b):
    def body(p_ref, deg_ref, b_ref, o_ref):
        deg = deg_ref[0] + deg_ref[1] + 1.0
        dis = lax.rsqrt(deg)
        s = p_ref[0] + p_ref[1]
        o_ref[...] = jnp.tanh(s * dis + b_ref[...][None, :])

    return pl.pallas_call(
        body,
        grid=(N // FB,),
        in_specs=[
            pl.BlockSpec((NC, FB, D), lambda i: (0, i, 0)),
            pl.BlockSpec((NC, FB, 1), lambda i: (0, i, 0)),
            pl.BlockSpec((D,), lambda i: (0,)),
        ],
        out_specs=pl.BlockSpec((FB, D), lambda i: (i, 0)),
        out_shape=jax.ShapeDtypeStruct((N, D), jnp.float32),
    )(parts, degs3, b)


def kernel(x, edge_index, W, b):
    ei = edge_index.astype(jnp.int32)
    x_pad = jnp.pad(x, ((0, NPAD - N), (0, 0)))

    degs = _deg_kernel(ei)
    g = _transform(x_pad, W, degs)
    parts = _prop_kernel(g, ei)
    return _finalize(parts, degs[:, :N, None], b)


# speedup vs baseline: 1.2208x; 1.0075x over previous
import functools

import jax
import jax.numpy as jnp
from jax import lax
from jax.experimental import pallas as pl
from jax.experimental.pallas import tpu as pltpu
from jax.experimental.pallas import tpu_sc as plsc

N = 10000
E = 320000
D = 128
NPAD = 10240
NC = 2
NS = 16
NW = NC * NS
BB = 128
NB = 78
EPW = NB * BB
XB = E // BB - NW * NB
XBASE = NW * EPW
ZPT = NPAD // NS
RB = 2048
FB = 2000
DEG_SLOTS = 16
DEG_WIN = 8

_mesh = plsc.VectorSubcoreMesh(core_axis_name="c", subcore_axis_name="s")


@functools.partial(
    pl.kernel,
    out_type=jax.ShapeDtypeStruct((NC, NPAD), jnp.float32),
    mesh=_mesh,
    scratch_types=[
        pltpu.VMEM((2, EPW), jnp.int32),
        pltpu.VMEM((DEG_SLOTS, BB), jnp.int32),
        pltpu.VMEM((2, BB), jnp.int32),
        pltpu.VMEM((BB,), jnp.float32),
        pltpu.VMEM((ZPT,), jnp.float32),
        pltpu.VMEM_SHARED((NPAD,), jnp.float32),
        pltpu.SemaphoreType.DMA,
        pltpu.SemaphoreType.DMA,
    ],
)
def _deg_kernel(ei_hbm, deg_out, idv, slots, ixt, ones_v, zb_v, sdeg,
                sem_p, sem_s):
    cid = lax.axis_index("c")
    sid = lax.axis_index("s")
    wid = sid * NC + cid
    base = wid * EPW

    pltpu.async_copy(ei_hbm.at[:, pl.ds(base, EPW)], idv, sem_p)

    def fill(i, _):
        ones_v[pl.ds(i * 16, 16)] = jnp.full((16,), 1.0, jnp.float32)
        return 0
    lax.fori_loop(0, BB // 16, fill, 0)

    def fill0(i, _):
        zb_v[pl.ds(i * 16, 16)] = jnp.zeros((16,), jnp.float32)
        return 0
    lax.fori_loop(0, ZPT // 16, fill0, 0)

    pltpu.sync_copy(zb_v, sdeg.at[pl.ds(sid * ZPT, ZPT)])
    pltpu.make_async_copy(ei_hbm.at[:, pl.ds(base, EPW)], idv, sem_p).wait()
    plsc.subcore_barrier()

    def scatter_wait():
        pltpu.make_async_copy(ones_v, sdeg.at[slots.at[0]], sem_s).wait()

    def step(i, _):
        b = lax.rem(i, DEG_SLOTS)
        for j in range(BB // 16):
            slots[b, pl.ds(j * 16, 16)] = idv[1, pl.ds(i * BB + j * 16, 16)]
        pltpu.async_copy(ones_v, sdeg.at[slots.at[b]], sem_s, add=True)

        @pl.when(i >= DEG_WIN)
        def _():
            scatter_wait()
        return 0
    lax.fori_loop(0, NB, step, 0)
    lax.fori_loop(0, DEG_WIN, lambda i, _: (scatter_wait(), 0)[1], 0)

    @pl.when(wid < XB)
    def _():
        pltpu.sync_copy(ei_hbm.at[:, pl.ds(XBASE + wid * BB, BB)], ixt)
        pltpu.async_copy(ones_v, sdeg.at[ixt.at[1]], sem_s, add=True)
        pltpu.make_async_copy(ones_v, sdeg.at[ixt.at[1]], sem_s).wait()

    plsc.subcore_barrier()
    pltpu.sync_copy(sdeg.at[pl.ds(sid * ZPT, ZPT)],
                    deg_out.at[cid, pl.ds(sid * ZPT, ZPT)])


@functools.partial(
    pl.kernel,
    out_type=jax.ShapeDtypeStruct((NC, NPAD, D), jnp.float32),
    mesh=_mesh,
    scratch_types=[
        pltpu.VMEM((8, BB), jnp.int32),
        pltpu.VMEM((2, BB, D), jnp.float32),
        pltpu.VMEM((2, BB), jnp.int32),
        pltpu.VMEM_SHARED((NPAD, D), jnp.float32),
        pltpu.SemaphoreType.DMA,
        pltpu.SemaphoreType.DMA,
        pltpu.SemaphoreType.DMA,
    ],
)
def _prop_kernel(g_hbm, ei_hbm, parts_out, i2, rows, ixt, sacc,
                 sem_i, sem_g, sem_s):
    cid = lax.axis_index("c")
    sid = lax.axis_index("s")
    wid = sid * NC + cid
    base = wid * EPW

    def i2_start(i):
        s = lax.rem(i, 4)
        pltpu.async_copy(ei_hbm.at[:, pl.ds(base + i * BB, BB)],
                         i2.at[pl.ds(2 * s, 2)], sem_i)

    def i2_wait():
        pltpu.make_async_copy(ei_hbm.at[:, pl.ds(base, BB)],
                              i2.at[pl.ds(0, 2)], sem_i).wait()

    def gather_start(i, b):
        pltpu.async_copy(g_hbm.at[i2.at[2 * lax.rem(i, 4)]], rows.at[b],
                         sem_g)

    def gather_wait():
        pltpu.make_async_copy(g_hbm.at[i2.at[0]], rows.at[0], sem_g).wait()

    def scatter_start(i, b):
        pltpu.async_copy(rows.at[b], sacc.at[i2.at[2 * lax.rem(i, 4) + 1]],
                         sem_s, add=True)

    def scatter_wait():
        pltpu.make_async_copy(rows.at[0], sacc.at[i2.at[1]], sem_s).wait()

    i2_start(0)
    i2_start(1)

    zbase = sid * ZPT

    @pl.when(cid == 0)
    def _():
        for off in (0, 128, 256, 384, 512):
            pltpu.async_copy(g_hbm.at[pl.ds(zbase + off, BB)],
                             sacc.at[pl.ds(zbase + off, BB)], sem_g)
        for off in (0, 128, 256, 384, 512):
            pltpu.make_async_copy(g_hbm.at[pl.ds(zbase + off, BB)],
                                  sacc.at[pl.ds(zbase + off, BB)],
                                  sem_g).wait()

    @pl.when(cid != 0)
    def _():
        def fill0(i, _):
            rows[1, i // 8, pl.ds((i % 8) * 16, 16)] = jnp.zeros((16,),
                                                                 jnp.float32)
            return 0
        lax.fori_loop(0, BB * (D // 16), fill0, 0)
        for off in (0, 128, 256, 384, 512):
            pltpu.async_copy(rows.at[1], sacc.at[pl.ds(zbase + off, BB)],
                             sem_g)
        for off in (0, 128, 256, 384, 512):
            pltpu.make_async_copy(rows.at[1],
                                  sacc.at[pl.ds(zbase + off, BB)],
                                  sem_g).wait()
    plsc.subcore_barrier()

    i2_wait()
    gather_start(0, 0)
    gather_wait()
    scatter_start(0, 0)
    i2_start(2)
    i2_wait()
    gather_start(1, 1)

    def step(i, _):
        b = lax.rem(i, 2)
        gather_wait()
        scatter_start(i, b)
        scatter_wait()
        i2_start(i + 2)
        i2_wait()
        gather_start(i + 1, 1 - b)
        return 0
    lax.fori_loop(1, NB - 2, step, 0)

    gather_wait()
    scatter_start(NB - 2, lax.rem(NB - 2, 2))
    scatter_wait()
    i2_wait()
    gather_start(NB - 1, lax.rem(NB - 1, 2))
    gather_wait()
    scatter_start(NB - 1, lax.rem(NB - 1, 2))
    scatter_wait()
    scatter_wait()

    @pl.when(wid < XB)
    def _():
        pltpu.sync_copy(ei_hbm.at[:, pl.ds(XBASE + wid * BB, BB)], ixt)
        pltpu.async_copy(g_hbm.at[ixt.at[0]], rows.at[0], sem_g)
        pltpu.make_async_copy(g_hbm.at[ixt.at[0]], rows.at[0], sem_g).wait()
        pltpu.async_copy(rows.at[0], sacc.at[ixt.at[1]], sem_s, add=True)
        pltpu.make_async_copy(rows.at[0], sacc.at[ixt.at[1]], sem_s).wait()

    plsc.subcore_barrier()
    pltpu.sync_copy(sacc.at[pl.ds(sid * ZPT, ZPT)],
                    parts_out.at[cid, pl.ds(sid * ZPT, ZPT)])


def _transform(x_pad, W, degs):
    def body(x_ref, w_ref, deg_ref, g_ref):
        deg = deg_ref[0, :] + deg_ref[1, :] + 1.0
        dis = lax.rsqrt(deg)
        h = jnp.dot(x_ref[...], w_ref[...], preferred_element_type=jnp.float32)
        g_ref[...] = h * dis[:, None]

    return pl.pallas_call(
        body,
        grid=(NPAD // RB,),
        in_specs=[
            pl.BlockSpec((RB, D), lambda i: (i, 0)),
            pl.BlockSpec((D, D), lambda i: (0, 0)),
            pl.BlockSpec((NC, RB), lambda i: (0, i)),
        ],
        out_specs=pl.BlockSpec((RB, D), lambda i: (i, 0)),
        out_shape=jax.ShapeDtypeStruct((NPAD, D), jnp.float32),
    )(x_pad, W, degs)


def _finalize(parts, degs3, b):
    def body(p_ref, deg_ref, b_ref, o_ref):
        deg = deg_ref[0] + deg_ref[1] + 1.0
        dis = lax.rsqrt(deg)
        s = p_ref[0] + p_ref[1]
        o_ref[...] = jnp.tanh(s * dis + b_ref[...][None, :])

    return pl.pallas_call(
        body,
        grid=(N // FB,),
        in_specs=[
            pl.BlockSpec((NC, FB, D), lambda i: (0, i, 0)),
            pl.BlockSpec((NC, FB, 1), lambda i: (0, i, 0)),
            pl.BlockSpec((D,), lambda i: (0,)),
        ],
        out_specs=pl.BlockSpec((FB, D), lambda i: (i, 0)),
        out_shape=jax.ShapeDtypeStruct((N, D), jnp.float32),
    )(parts, degs3, b)


def kernel(x, edge_index, W, b):
    ei = edge_index.astype(jnp.int32)
    x_pad = jnp.pad(x, ((0, NPAD - N), (0, 0)))

    degs = _deg_kernel(ei)
    g = _transform(x_pad, W, degs)
    parts = _prop_kernel(g, ei)
    return _finalize(parts, degs[:, :N, None], b)
